# baseline (speedup 1.0000x reference)
import functools

import jax
import jax.numpy as jnp
from jax import lax
from jax.experimental import pallas as pl
from jax.experimental.pallas import tpu as pltpu
from jax.experimental.pallas import tpu_sc as plsc

_BS = 64
_NCROPS = 5
_T = 256
_D = 1024
_K = 3
_ALPHA = 0.0001
_MARGIN = 100.0
_B2 = _BS // 2

_NEG_INF = float("-inf")


_VPB = 4


def _mag_body(x_ref, o_ref):
    x = x_ref[0]
    ones = jnp.ones((_D, 1), jnp.float32)
    s = jax.lax.dot_general(
        x * x, ones, (((1,), (0,)), ((), ())),
        preferred_element_type=jnp.float32,
    )
    v = jnp.sqrt(s) * (1.0 / _NCROPS)
    for b in range(_VPB):
        acc = v[b * _NCROPS * _T : b * _NCROPS * _T + _T]
        for c in range(1, _NCROPS):
            lo = b * _NCROPS * _T + c * _T
            acc = acc + v[lo : lo + _T]
        o_ref[0, b] = acc[:, 0]


def _feature_magnitudes(feats3):
    return pl.pallas_call(
        _mag_body,
        grid=(_BS // _VPB,),
        in_specs=[pl.BlockSpec((1, _VPB * _NCROPS * _T, _D), lambda b: (b, 0, 0))],
        out_specs=pl.BlockSpec((1, _VPB, _T), lambda b: (b, 0, 0)),
        out_shape=jax.ShapeDtypeStruct((_BS // _VPB, _VPB, _T), jnp.float32),
    )(feats3)


_MESH = plsc.VectorSubcoreMesh(core_axis_name="c", subcore_axis_name="s")


@functools.partial(
    pl.kernel,
    mesh=_MESH,
    out_type=[
        jax.ShapeDtypeStruct((_BS * _NCROPS * 16,), jnp.float32),
        jax.ShapeDtypeStruct((_BS * 16,), jnp.float32),
    ],
    scratch_types=[
        pltpu.VMEM((_T,), jnp.float32),
        pltpu.VMEM((_NCROPS * _T,), jnp.float32),
        pltpu.VMEM((16,), jnp.int32),
        pltpu.VMEM((32,), jnp.float32),
        pltpu.VMEM((32,), jnp.int32),
        pltpu.VMEM((16, _D), jnp.float32),
        pltpu.VMEM((_NCROPS * 16,), jnp.float32),
        pltpu.VMEM((16,), jnp.float32),
        pltpu.SemaphoreType.DMA,
    ],
)
def _sc_topk_gather(fm_hbm, sc_hbm, feats_hbm, n2_out, lg_out,
                    fm_v, sc_v, idx_v, vrot, irot, rows_v, n2_s, lg_s, sem):
    wid = lax.axis_index("s") * 2 + lax.axis_index("c")
    lane = lax.iota(jnp.int32, 16)
    zeros_i = jnp.zeros((16,), jnp.int32)

    for vb in range(2):
        b = wid * 2 + vb

        pltpu.sync_copy(fm_hbm.at[pl.ds(b * _T, _T)], fm_v)
        pltpu.sync_copy(sc_hbm.at[pl.ds(b * _NCROPS * _T, _NCROPS * _T)], sc_v)

        v1 = v2 = v3 = jnp.full((16,), _NEG_INF, jnp.float32)
        i1 = i2 = i3 = zeros_i
        for k in range(16):
            x = fm_v[pl.ds(k * 16, 16)]
            t = lane + (k * 16)
            b1 = x > v1
            b2 = x > v2
            b3 = x > v3
            nv1 = jnp.where(b1, x, v1)
            ni1 = jnp.where(b1, t, i1)
            nv2 = jnp.where(b1, v1, jnp.where(b2, x, v2))
            ni2 = jnp.where(b1, i1, jnp.where(b2, t, i2))
            nv3 = jnp.where(b2, v2, jnp.where(b3, x, v3))
            ni3 = jnp.where(b2, i2, jnp.where(b3, t, i3))
            v1, v2, v3, i1, i2, i3 = nv1, nv2, nv3, ni1, ni2, ni3

        t_splats = []
        for _ in range(_K):
            rv, ri = v1, i1
            for sh in (1, 2, 4, 8):
                vrot[pl.ds(0, 16)] = rv
                vrot[pl.ds(16, 16)] = rv
                irot[pl.ds(0, 16)] = ri
                irot[pl.ds(16, 16)] = ri
                pv = vrot[pl.ds(sh, 16)]
                pi = irot[pl.ds(sh, 16)]
                take = (pv > rv) | ((pv == rv) & (pi < ri))
                rv = jnp.where(take, pv, rv)
                ri = jnp.where(take, pi, ri)
            t_splat = ri
            t_splats.append(t_splat)
            onehot = i1 == t_splat
            v1 = jnp.where(onehot, v2, v1)
            i1 = jnp.where(onehot, i2, i1)
            v2 = jnp.where(onehot, v3, v2)
            i2 = jnp.where(onehot, i3, i2)
        t1, t2, t3 = t_splats

        c_vec = jnp.zeros((16,), jnp.int32)
        for thr in (3, 6, 9, 12):
            c_vec = jnp.where(lane >= thr, c_vec + 1, c_vec)
        k_vec = lane - c_vec * 3
        t_vec = jnp.where(k_vec == 0, t1, jnp.where(k_vec == 1, t2, t3))

        acc = jnp.zeros((16,), jnp.float32)
        zf = jnp.zeros((16,), jnp.float32)
        for k in range(16):
            t_pos = lane + (k * 16)
            cs = sc_v[pl.ds(k * 16, 16)]
            for c in range(1, _NCROPS):
                cs = cs + sc_v[pl.ds(c * _T + k * 16, 16)]
            acc = (acc
                   + jnp.where(t_pos == t1, cs, zf)
                   + jnp.where(t_pos == t2, cs, zf)
                   + jnp.where(t_pos == t3, cs, zf))
        lg_s[...] = acc
        pltpu.sync_copy(lg_s, lg_out.at[pl.ds(b * 16, 16)])

        idx_v[...] = (b * _NCROPS + c_vec) * _T + t_vec
        pltpu.async_copy(feats_hbm.at[idx_v], rows_v, sem).wait()

        def body(j, accs):
            out = []
            for c in range(_NCROPS):
                s = (rows_v[3 * c, pl.ds(j * 16, 16)]
                     + rows_v[3 * c + 1, pl.ds(j * 16, 16)]
                     + rows_v[3 * c + 2, pl.ds(j * 16, 16)])
                out.append(accs[c] + s * s)
            return tuple(out)

        accs = lax.fori_loop(0, _D // 16, body,
                             tuple(jnp.zeros((16,), jnp.float32) for _ in range(_NCROPS)))
        for c in range(_NCROPS):
            n2_s[pl.ds(c * 16, 16)] = accs[c]
        pltpu.sync_copy(n2_s, n2_out.at[pl.ds(b * _NCROPS * 16, _NCROPS * 16)])


def _loss_body(n2_ref, lg_ref, lab_ref, o_ref):
    l2 = jnp.sqrt(jnp.sum(n2_ref[...], axis=1) * (1.0 / 9.0))
    l_abn = jnp.abs(_MARGIN - l2[: _B2 * _NCROPS])
    l_nor = l2[_B2 * _NCROPS:]
    loss_mgnt = jnp.mean((l_abn + l_nor) ** 2)
    x = jnp.sum(lg_ref[...], axis=1) * (1.0 / (_NCROPS * _K))
    y = lab_ref[0]
    sp = jnp.maximum(x, 0.0) + jnp.log1p(jnp.exp(-jnp.abs(x)))
    loss_scor = jnp.mean(sp - x * y)
    o_ref[...] = jnp.stack([_ALPHA * loss_mgnt, loss_scor]).reshape(1, 2)


def _final_losses(n2s, lg, labels):
    return pl.pallas_call(
        _loss_body,
        in_specs=[
            pl.BlockSpec((_BS * _NCROPS, 16), lambda: (0, 0)),
            pl.BlockSpec((_BS, 16), lambda: (0, 0)),
            pl.BlockSpec((1, _BS), lambda: (0, 0)),
        ],
        out_specs=pl.BlockSpec((1, 2), lambda: (0, 0)),
        out_shape=jax.ShapeDtypeStruct((1, 2), jnp.float32),
    )(n2s, lg, labels)


def kernel(feats, scores, labels):
    feats3 = feats.reshape(_BS // _VPB, _VPB * _NCROPS * _T, _D)
    fmgnt = _feature_magnitudes(feats3)

    n2_flat, lg_flat = _sc_topk_gather(
        fmgnt.reshape(_BS * _T),
        scores.reshape(_BS * _NCROPS * _T),
        feats.reshape(_BS * _NCROPS * _T, _D),
    )
    n2s = n2_flat.reshape(_BS * _NCROPS, 16)
    lg = lg_flat.reshape(_BS, 16)

    out = _final_losses(n2s, lg, labels.reshape(1, _BS))
    return out.reshape(2)


# speedup vs baseline: 1.0451x; 1.0451x over previous
import functools

import jax
import jax.numpy as jnp
from jax import lax
from jax.experimental import pallas as pl
from jax.experimental.pallas import tpu as pltpu
from jax.experimental.pallas import tpu_sc as plsc

_BS = 64
_NCROPS = 5
_T = 256
_D = 1024
_K = 3
_ALPHA = 0.0001
_MARGIN = 100.0
_B2 = _BS // 2

_NEG_INF = float("-inf")


_VPB = 2


def _mag_body(x_ref, o_ref):
    x = x_ref[0]
    s = jnp.sum(x * x, axis=1, keepdims=True)
    v = jnp.sqrt(s) * (1.0 / _NCROPS)
    for b in range(_VPB):
        acc = v[b * _NCROPS * _T : b * _NCROPS * _T + _T]
        for c in range(1, _NCROPS):
            lo = b * _NCROPS * _T + c * _T
            acc = acc + v[lo : lo + _T]
        o_ref[0, b] = acc[:, 0]


def _feature_magnitudes(feats3):
    return pl.pallas_call(
        _mag_body,
        grid=(_BS // _VPB,),
        in_specs=[pl.BlockSpec((1, _VPB * _NCROPS * _T, _D), lambda b: (b, 0, 0))],
        out_specs=pl.BlockSpec((1, _VPB, _T), lambda b: (b, 0, 0)),
        out_shape=jax.ShapeDtypeStruct((_BS // _VPB, _VPB, _T), jnp.float32),
    )(feats3)


_MESH = plsc.VectorSubcoreMesh(core_axis_name="c", subcore_axis_name="s")


@functools.partial(
    pl.kernel,
    mesh=_MESH,
    out_type=[
        jax.ShapeDtypeStruct((_BS * _NCROPS * 16,), jnp.float32),
        jax.ShapeDtypeStruct((_BS * 16,), jnp.float32),
    ],
    scratch_types=[
        pltpu.VMEM((_T,), jnp.float32),
        pltpu.VMEM((_NCROPS * _T,), jnp.float32),
        pltpu.VMEM((16,), jnp.int32),
        pltpu.VMEM((32,), jnp.float32),
        pltpu.VMEM((32,), jnp.int32),
        pltpu.VMEM((16, _D), jnp.float32),
        pltpu.VMEM((_NCROPS * 16,), jnp.float32),
        pltpu.VMEM((16,), jnp.float32),
        pltpu.SemaphoreType.DMA,
    ],
)
def _sc_topk_gather(fm_hbm, sc_hbm, feats_hbm, n2_out, lg_out,
                    fm_v, sc_v, idx_v, vrot, irot, rows_v, n2_s, lg_s, sem):
    wid = lax.axis_index("s") * 2 + lax.axis_index("c")
    lane = lax.iota(jnp.int32, 16)
    zeros_i = jnp.zeros((16,), jnp.int32)

    for vb in range(2):
        b = wid * 2 + vb

        pltpu.sync_copy(fm_hbm.at[pl.ds(b * _T, _T)], fm_v)
        pltpu.sync_copy(sc_hbm.at[pl.ds(b * _NCROPS * _T, _NCROPS * _T)], sc_v)

        v1 = v2 = v3 = jnp.full((16,), _NEG_INF, jnp.float32)
        i1 = i2 = i3 = zeros_i
        for k in range(16):
            x = fm_v[pl.ds(k * 16, 16)]
            t = lane + (k * 16)
            b1 = x > v1
            b2 = x > v2
            b3 = x > v3
            nv1 = jnp.where(b1, x, v1)
            ni1 = jnp.where(b1, t, i1)
            nv2 = jnp.where(b1, v1, jnp.where(b2, x, v2))
            ni2 = jnp.where(b1, i1, jnp.where(b2, t, i2))
            nv3 = jnp.where(b2, v2, jnp.where(b3, x, v3))
            ni3 = jnp.where(b2, i2, jnp.where(b3, t, i3))
            v1, v2, v3, i1, i2, i3 = nv1, nv2, nv3, ni1, ni2, ni3

        t_splats = []
        for _ in range(_K):
            rv, ri = v1, i1
            for sh in (1, 2, 4, 8):
                vrot[pl.ds(0, 16)] = rv
                vrot[pl.ds(16, 16)] = rv
                irot[pl.ds(0, 16)] = ri
                irot[pl.ds(16, 16)] = ri
                pv = vrot[pl.ds(sh, 16)]
                pi = irot[pl.ds(sh, 16)]
                take = (pv > rv) | ((pv == rv) & (pi < ri))
                rv = jnp.where(take, pv, rv)
                ri = jnp.where(take, pi, ri)
            t_splat = ri
            t_splats.append(t_splat)
            onehot = i1 == t_splat
            v1 = jnp.where(onehot, v2, v1)
            i1 = jnp.where(onehot, i2, i1)
            v2 = jnp.where(onehot, v3, v2)
            i2 = jnp.where(onehot, i3, i2)
        t1, t2, t3 = t_splats

        c_vec = jnp.zeros((16,), jnp.int32)
        for thr in (3, 6, 9, 12):
            c_vec = jnp.where(lane >= thr, c_vec + 1, c_vec)
        k_vec = lane - c_vec * 3
        t_vec = jnp.where(k_vec == 0, t1, jnp.where(k_vec == 1, t2, t3))

        acc = jnp.zeros((16,), jnp.float32)
        zf = jnp.zeros((16,), jnp.float32)
        for k in range(16):
            t_pos = lane + (k * 16)
            cs = sc_v[pl.ds(k * 16, 16)]
            for c in range(1, _NCROPS):
                cs = cs + sc_v[pl.ds(c * _T + k * 16, 16)]
            acc = (acc
                   + jnp.where(t_pos == t1, cs, zf)
                   + jnp.where(t_pos == t2, cs, zf)
                   + jnp.where(t_pos == t3, cs, zf))
        lg_s[...] = acc
        pltpu.sync_copy(lg_s, lg_out.at[pl.ds(b * 16, 16)])

        idx_v[...] = (b * _NCROPS + c_vec) * _T + t_vec
        pltpu.async_copy(feats_hbm.at[idx_v], rows_v, sem).wait()

        def body(j, accs):
            out = []
            for c in range(_NCROPS):
                s = (rows_v[3 * c, pl.ds(j * 16, 16)]
                     + rows_v[3 * c + 1, pl.ds(j * 16, 16)]
                     + rows_v[3 * c + 2, pl.ds(j * 16, 16)])
                out.append(accs[c] + s * s)
            return tuple(out)

        accs = lax.fori_loop(0, _D // 16, body,
                             tuple(jnp.zeros((16,), jnp.float32) for _ in range(_NCROPS)))
        for c in range(_NCROPS):
            n2_s[pl.ds(c * 16, 16)] = accs[c]
        pltpu.sync_copy(n2_s, n2_out.at[pl.ds(b * _NCROPS * 16, _NCROPS * 16)])


def _loss_body(n2_ref, lg_ref, lab_ref, o_ref):
    l2 = jnp.sqrt(jnp.sum(n2_ref[...], axis=1) * (1.0 / 9.0))
    l_abn = jnp.abs(_MARGIN - l2[: _B2 * _NCROPS])
    l_nor = l2[_B2 * _NCROPS:]
    loss_mgnt = jnp.mean((l_abn + l_nor) ** 2)
    x = jnp.sum(lg_ref[...], axis=1) * (1.0 / (_NCROPS * _K))
    y = lab_ref[0]
    sp = jnp.maximum(x, 0.0) + jnp.log1p(jnp.exp(-jnp.abs(x)))
    loss_scor = jnp.mean(sp - x * y)
    o_ref[...] = jnp.stack([_ALPHA * loss_mgnt, loss_scor]).reshape(1, 2)


def _final_losses(n2s, lg, labels):
    return pl.pallas_call(
        _loss_body,
        in_specs=[
            pl.BlockSpec((_BS * _NCROPS, 16), lambda: (0, 0)),
            pl.BlockSpec((_BS, 16), lambda: (0, 0)),
            pl.BlockSpec((1, _BS), lambda: (0, 0)),
        ],
        out_specs=pl.BlockSpec((1, 2), lambda: (0, 0)),
        out_shape=jax.ShapeDtypeStruct((1, 2), jnp.float32),
    )(n2s, lg, labels)


def kernel(feats, scores, labels):
    feats3 = feats.reshape(_BS // _VPB, _VPB * _NCROPS * _T, _D)
    fmgnt = _feature_magnitudes(feats3)

    n2_flat, lg_flat = _sc_topk_gather(
        fmgnt.reshape(_BS * _T),
        scores.reshape(_BS * _NCROPS * _T),
        feats.reshape(_BS * _NCROPS * _T, _D),
    )
    n2s = n2_flat.reshape(_BS * _NCROPS, 16)
    lg = lg_flat.reshape(_BS, 16)

    out = _final_losses(n2s, lg, labels.reshape(1, _BS))
    return out.reshape(2)
